# Initial kernel scaffold; baseline (speedup 1.0000x reference)
import jax
import jax.numpy as jnp
from jax.experimental import pallas as pl


def kernel(x, params, ww_src, ww_dst, wwr_src, wwr_dst, wd_src, wd_dst, wdr_src, wdr_dst):
    raise NotImplementedError("write your pallas kernel here")


# speedup vs baseline: 6.6583x; 6.6583x over previous
import functools

import jax
import jax.numpy as jnp
from jax.experimental import pallas as pl

_NW = 10000
_ND = 15000
_D = 128
_H = 4
_F = 128
_PW = 10240
_PD = 15360
_B = 512


def _proj_body(x_ref, wt_ref, al_ref, ar_ref, h_ref, el_ref, er_ref, *, R):
    x = x_ref[...]
    h = jnp.dot(x, wt_ref[...], preferred_element_type=jnp.float32)
    h_ref[...] = h
    b = x.shape[0]
    hr = h.reshape(b, R, _H, _F)
    al = al_ref[...].reshape(1, R, _H, _F)
    ar = ar_ref[...].reshape(1, R, _H, _F)
    el_ref[...] = jnp.sum(hr * al, axis=-1).reshape(b, R * _H)
    er_ref[...] = jnp.sum(hr * ar, axis=-1).reshape(b, R * _H)


def _project(xp, wt, al, ar, R):
    n = xp.shape[0]
    k = R * _H * _F
    return pl.pallas_call(
        functools.partial(_proj_body, R=R),
        grid=(n // _B,),
        in_specs=[
            pl.BlockSpec((_B, _D), lambda i: (i, 0)),
            pl.BlockSpec((_D, k), lambda i: (0, 0)),
            pl.BlockSpec((1, k), lambda i: (0, 0)),
            pl.BlockSpec((1, k), lambda i: (0, 0)),
        ],
        out_specs=[
            pl.BlockSpec((_B, k), lambda i: (i, 0)),
            pl.BlockSpec((_B, R * _H), lambda i: (i, 0)),
            pl.BlockSpec((_B, R * _H), lambda i: (i, 0)),
        ],
        out_shape=[
            jax.ShapeDtypeStruct((n, k), jnp.float32),
            jax.ShapeDtypeStruct((n, R * _H), jnp.float32),
            jax.ShapeDtypeStruct((n, R * _H), jnp.float32),
        ],
    )(xp, wt, al.reshape(1, k), ar.reshape(1, k))


def _head_body(x_ref, w_ref, b_ref, o_ref):
    y = jnp.sum(x_ref[...] * w_ref[...], axis=1, keepdims=True) + b_ref[0, 0]
    o_ref[...] = jax.nn.sigmoid(y)


def _head(xdp, w, b):
    n = xdp.shape[0]
    return pl.pallas_call(
        _head_body,
        grid=(n // _B,),
        in_specs=[
            pl.BlockSpec((_B, _F), lambda i: (i, 0)),
            pl.BlockSpec((1, _F), lambda i: (0, 0)),
            pl.BlockSpec((1, 1), lambda i: (0, 0)),
        ],
        out_specs=pl.BlockSpec((_B, 1), lambda i: (i, 0)),
        out_shape=jax.ShapeDtypeStruct((n, 1), jnp.float32),
    )(xdp, w, b.reshape(1, 1))


def _edge_msg(el_src, er_dst, h_src, src, dst, n_dst):
    e = jax.nn.leaky_relu(el_src[src] + er_dst[dst], negative_slope=0.2)
    m = jax.ops.segment_max(e, dst, num_segments=n_dst)
    m = jnp.where(jnp.isfinite(m), m, 0.0)
    ex = jnp.exp(e - m[dst])
    s = jax.ops.segment_sum(ex, dst, num_segments=n_dst)
    alpha = ex / (s[dst] + 1e-16)
    return jnp.einsum('eh,ehf->ef', alpha, h_src[src])


def _bias_sum(p):
    return p['bias'].reshape(_H, _F).sum(axis=0)


def _layer(xw, xd, rel, ww_src, ww_dst, wwr_src, wwr_dst,
           wd_src, wd_dst, wdr_src, wdr_dst):
    word_rels = ['word2word', 'word2wordr', 'word2document', 'word2documentr']
    doc_rels = ['word2document', 'word2documentr']

    wt_w = jnp.concatenate([rel[n]['W'].T for n in word_rels], axis=1)
    al_w = jnp.concatenate([rel[n]['attn_l'].reshape(-1) for n in word_rels])
    ar_w = jnp.concatenate([rel[n]['attn_r'].reshape(-1) for n in word_rels])
    wt_d = jnp.concatenate([rel[n]['W'].T for n in doc_rels], axis=1)
    al_d = jnp.concatenate([rel[n]['attn_l'].reshape(-1) for n in doc_rels])
    ar_d = jnp.concatenate([rel[n]['attn_r'].reshape(-1) for n in doc_rels])

    xwp = jnp.pad(xw, ((0, _PW - _NW), (0, 0)))
    xdp = jnp.pad(xd, ((0, _PD - _ND), (0, 0)))

    h_w, el_w, er_w = _project(xwp, wt_w, al_w, ar_w, R=4)
    h_d, el_d, er_d = _project(xdp, wt_d, al_d, ar_d, R=2)

    hw = h_w[:_NW].reshape(_NW, 4, _H, _F)
    el_w = el_w[:_NW].reshape(_NW, 4, _H)
    er_w = er_w[:_NW].reshape(_NW, 4, _H)
    hd = h_d[:_ND].reshape(_ND, 2, _H, _F)
    el_d = el_d[:_ND].reshape(_ND, 2, _H)
    er_d = er_d[:_ND].reshape(_ND, 2, _H)

    msg_ww = _edge_msg(el_w[:, 0], er_w[:, 0], hw[:, 0], ww_src, ww_dst, _NW)
    msg_wwr = _edge_msg(el_w[:, 1], er_w[:, 1], hw[:, 1], wwr_src, wwr_dst, _NW)
    msg_wdr = _edge_msg(el_d[:, 1], er_w[:, 3], hd[:, 1], wdr_src, wdr_dst, _NW)
    msg_wd = _edge_msg(el_w[:, 2], er_d[:, 0], hw[:, 2], wd_src, wd_dst, _ND)

    xw_new = jax.ops.segment_sum(
        jnp.concatenate([msg_ww, msg_wwr, msg_wdr], axis=0),
        jnp.concatenate([ww_dst, wwr_dst, wdr_dst], axis=0),
        num_segments=_NW)
    xw_new = xw_new + (_bias_sum(rel['word2word'])
                       + _bias_sum(rel['word2wordr'])
                       + _bias_sum(rel['word2documentr']))[None, :]
    xd_new = jax.ops.segment_sum(msg_wd, wd_dst, num_segments=_ND)
    xd_new = xd_new + _bias_sum(rel['word2document'])[None, :]
    return jax.nn.relu(xw_new), jax.nn.relu(xd_new)


def kernel(x, params, ww_src, ww_dst, wwr_src, wwr_dst, wd_src, wd_dst, wdr_src, wdr_dst):
    xw = x
    xd = params['doc_emb']
    for li in range(2):
        xw, xd = _layer(xw, xd, params['layers'][li],
                        ww_src, ww_dst, wwr_src, wwr_dst,
                        wd_src, wd_dst, wdr_src, wdr_dst)
    xdp = jnp.pad(xd, ((0, _PD - _ND), (0, 0)))
    out = _head(xdp, params['lin1_w'], params['lin1_b'])
    return out[:_ND]
